# baseline (speedup 1.0000x reference)
import functools

import jax
import jax.numpy as jnp
from jax import lax
from jax.experimental import pallas as pl
from jax.experimental.pallas import tpu as pltpu
from jax.experimental.pallas import tpu_sc as plsc

N_CAT = 26
N_NUM = 13
VOCAB = 100000
D = 32
B = 16384
L_TOK = 1 + N_CAT + N_NUM

NC, NS, LANES = 2, 16, 16
NW = NC * NS
ROWS_W = B // NW
NB = 32
NCHUNK = ROWS_W // NB
IDX_PER_CHUNK = NB * N_CAT
GGRP = 8
GIDX = IDX_PER_CHUNK // GGRP
ROW_ELEMS = L_TOK * D
OUT_PER_CHUNK = NB * ROW_ELEMS


def _sc_body(xcat_hbm, xnum_hbm, table_hbm, w_hbm, nb_hbm, pos_hbm, cls_hbm,
             out_hbm,
             xcat_v, off_v, gat_v, out_v, xnum_v, pos_v, w_v, c_v, cls_v,
             sem):
    wid = lax.axis_index("s") * NC + lax.axis_index("c")

    pltpu.sync_copy(pos_hbm, pos_v)
    pltpu.sync_copy(w_hbm, w_v)
    pltpu.sync_copy(nb_hbm, c_v)
    pltpu.sync_copy(cls_hbm, cls_v)

    def off_body(i, carry):
        p = i * LANES + lax.iota(jnp.int32, LANES)
        off_v[pl.ds(i * LANES, LANES)] = (p % N_CAT) * VOCAB
        return carry
    lax.fori_loop(0, IDX_PER_CHUNK // LANES, off_body, 0)

    for j in range(N_NUM):
        for k in range(0, D, LANES):
            o = j * D + k
            c_v[pl.ds(o, LANES)] = c_v[pl.ds(o, LANES)] + pos_v[pl.ds((1 + N_CAT + j) * D + k, LANES)]
    for k in range(0, D, LANES):
        cls_v[pl.ds(k, LANES)] = cls_v[pl.ds(k, LANES)] + pos_v[pl.ds(k, LANES)]

    def chunk_body(c, carry):
        base_row = wid * ROWS_W + c * NB

        pltpu.sync_copy(xcat_hbm.at[pl.ds(base_row * N_CAT, IDX_PER_CHUNK)], xcat_v)
        pltpu.sync_copy(xnum_hbm.at[pl.ds(base_row * N_NUM, NB * N_NUM)],
                        xnum_v.at[pl.ds(0, NB * N_NUM)])

        def idx_body(i, cy):
            s = i * LANES
            xcat_v[pl.ds(s, LANES)] = xcat_v[pl.ds(s, LANES)] + off_v[pl.ds(s, LANES)]
            return cy
        lax.fori_loop(0, IDX_PER_CHUNK // LANES, idx_body, 0)

        copies = []
        for g in range(GGRP):
            src = table_hbm.at[xcat_v.at[pl.ds(g * GIDX, GIDX)]]
            dst = gat_v.at[pl.ds(g * GIDX, GIDX), :]
            copies.append(pltpu.async_copy(src, dst, sem))
        for cp in copies:
            cp.wait()

        for f in range(N_CAT):
            p0 = pos_v[pl.ds((1 + f) * D, LANES)]
            p1 = pos_v[pl.ds((1 + f) * D + LANES, LANES)]

            def cat_body(b, cy, f=f, p0=p0, p1=p1):
                r = b * N_CAT + f
                o = b * ROW_ELEMS + (1 + f) * D
                out_v[pl.ds(o, LANES)] = gat_v[r, pl.ds(0, LANES)] + p0
                out_v[pl.ds(o + LANES, LANES)] = gat_v[r, pl.ds(LANES, LANES)] + p1
                return cy
            lax.fori_loop(0, NB, cat_body, 0)

        for j in range(N_NUM):
            w0 = w_v[pl.ds(j * D, LANES)]
            w1 = w_v[pl.ds(j * D + LANES, LANES)]
            c0 = c_v[pl.ds(j * D, LANES)]
            c1 = c_v[pl.ds(j * D + LANES, LANES)]

            def num_body(b, cy, j=j, w0=w0, w1=w1, c0=c0, c1=c1):
                v = xnum_v[pl.ds(b * N_NUM + j, LANES)]
                sv = jnp.broadcast_to(v[0], (LANES,))
                o = b * ROW_ELEMS + (1 + N_CAT + j) * D
                out_v[pl.ds(o, LANES)] = sv * w0 + c0
                out_v[pl.ds(o + LANES, LANES)] = sv * w1 + c1
                return cy
            lax.fori_loop(0, NB, num_body, 0)

        def cls_body(b, cy):
            o = b * ROW_ELEMS
            out_v[pl.ds(o, LANES)] = cls_v[pl.ds(0, LANES)]
            out_v[pl.ds(o + LANES, LANES)] = cls_v[pl.ds(LANES, LANES)]
            return cy
        lax.fori_loop(0, NB, cls_body, 0)

        pltpu.sync_copy(out_v, out_hbm.at[pl.ds(base_row * ROW_ELEMS, OUT_PER_CHUNK)])
        return carry

    lax.fori_loop(0, NCHUNK, chunk_body, 0)


@jax.jit
def kernel(x_cat, x_num, cat_tables, num_W, num_b, feature_pos, cls):
    mesh = plsc.VectorSubcoreMesh(core_axis_name="c", subcore_axis_name="s")
    k = pl.kernel(
        _sc_body,
        out_type=jax.ShapeDtypeStruct((B * ROW_ELEMS,), jnp.float32),
        mesh=mesh,
        compiler_params=pltpu.CompilerParams(use_tc_tiling_on_sc=False),
        scratch_types=[
            pltpu.VMEM((IDX_PER_CHUNK,), jnp.int32),
            pltpu.VMEM((IDX_PER_CHUNK,), jnp.int32),
            pltpu.VMEM((IDX_PER_CHUNK, D), jnp.float32),
            pltpu.VMEM((OUT_PER_CHUNK,), jnp.float32),
            pltpu.VMEM((NB * N_NUM + LANES,), jnp.float32),
            pltpu.VMEM((L_TOK * D,), jnp.float32),
            pltpu.VMEM((N_NUM * D,), jnp.float32),
            pltpu.VMEM((N_NUM * D,), jnp.float32),
            pltpu.VMEM((D,), jnp.float32),
            pltpu.SemaphoreType.DMA,
        ],
    )
    out = k(
        x_cat.reshape(B * N_CAT),
        x_num.reshape(B * N_NUM),
        cat_tables.reshape(N_CAT * VOCAB, D),
        num_W.reshape(N_NUM * D),
        num_b.reshape(N_NUM * D),
        feature_pos.reshape(L_TOK * D),
        cls.reshape(D),
    )
    return out.reshape(B, L_TOK, D)


# speedup vs baseline: 1.0047x; 1.0047x over previous
import jax
import jax.numpy as jnp
from jax import lax
from jax.experimental import pallas as pl
from jax.experimental.pallas import tpu as pltpu
from jax.experimental.pallas import tpu_sc as plsc

N_CAT = 26
N_NUM = 13
VOCAB = 100000
D = 32
B = 16384
L_TOK = 1 + N_CAT + N_NUM

NC, NS, LANES = 2, 16, 16
NW = NC * NS
ROWS_W = B // NW
NB = 16
NCHUNK = ROWS_W // NB
IDX_PER_CHUNK = NB * N_CAT
GGRP = 4
GIDX = IDX_PER_CHUNK // GGRP
ROW_ELEMS = L_TOK * D
OUT_PER_CHUNK = NB * ROW_ELEMS
VPR = 2 * N_CAT


def _sc_body(xcat_hbm, xnum_hbm, table_hbm, w_hbm, nb_hbm, pos_hbm, cls_hbm,
             out_hbm,
             xcat0, xcat1, gat0, gat1, out0, out1, xnum0, xnum1,
             pos_v, w_v, c_v, cls_v, off_v,
             sem_in0, sem_in1, sem_g0, sem_g1, sem_o0, sem_o1):
    wid = lax.axis_index("s") * NC + lax.axis_index("c")
    row0 = wid * ROWS_W

    xcat = (xcat0, xcat1)
    gat = (gat0, gat1)
    outb = (out0, out1)
    xnum = (xnum0, xnum1)
    sem_in = (sem_in0, sem_in1)
    sem_g = (sem_g0, sem_g1)
    sem_o = (sem_o0, sem_o1)

    pltpu.sync_copy(pos_hbm, pos_v)
    pltpu.sync_copy(w_hbm, w_v)
    pltpu.sync_copy(nb_hbm, c_v)
    pltpu.sync_copy(cls_hbm, cls_v)

    for i in range(IDX_PER_CHUNK // LANES):
        p = i * LANES + lax.iota(jnp.int32, LANES)
        off_v[pl.ds(i * LANES, LANES)] = (p % N_CAT) * VOCAB

    for j in range(N_NUM):
        for k in range(0, D, LANES):
            o = j * D + k
            c_v[pl.ds(o, LANES)] = c_v[pl.ds(o, LANES)] + \
                pos_v[pl.ds((1 + N_CAT + j) * D + k, LANES)]
    for k in range(0, D, LANES):
        cls_v[pl.ds(k, LANES)] = cls_v[pl.ds(k, LANES)] + pos_v[pl.ds(k, LANES)]

    def start_in(c, p):
        base = row0 + c * NB
        pltpu.async_copy(xcat_hbm.at[pl.ds(base * N_CAT, IDX_PER_CHUNK)],
                         xcat[p], sem_in[p])
        pltpu.async_copy(xnum_hbm.at[pl.ds(base * N_NUM, NB * N_NUM)],
                         xnum[p].at[pl.ds(0, NB * N_NUM)], sem_in[p])

    def fire_gather(p):
        pltpu.make_async_copy(xcat_hbm.at[pl.ds(0, IDX_PER_CHUNK)],
                              xcat[p], sem_in[p]).wait()
        pltpu.make_async_copy(xnum_hbm.at[pl.ds(0, NB * N_NUM)],
                              xnum[p].at[pl.ds(0, NB * N_NUM)], sem_in[p]).wait()
        for i in range(IDX_PER_CHUNK // LANES):
            s = i * LANES
            xcat[p][pl.ds(s, LANES)] = xcat[p][pl.ds(s, LANES)] + off_v[pl.ds(s, LANES)]
        for g in range(GGRP):
            pltpu.async_copy(table_hbm.at[xcat[p].at[pl.ds(g * GIDX, GIDX)]],
                             gat[p].at[pl.ds(g * GIDX, GIDX), :], sem_g[p])

    def finish(c, p, wait_out):
        if wait_out is not None:
            @pl.when(wait_out)
            def _():
                pltpu.make_async_copy(outb[p], out_hbm.at[pl.ds(0, OUT_PER_CHUNK)],
                                      sem_o[p]).wait()

        def nc_body(b, cy):
            o_row = b * ROW_ELEMS
            outb[p][pl.ds(o_row, LANES)] = cls_v[pl.ds(0, LANES)]
            outb[p][pl.ds(o_row + LANES, LANES)] = cls_v[pl.ds(LANES, LANES)]
            for j in range(N_NUM):
                v = xnum[p][pl.ds(b * N_NUM + j, LANES)]
                sv = jnp.broadcast_to(v[0], (LANES,))
                o = o_row + (1 + N_CAT + j) * D
                outb[p][pl.ds(o, LANES)] = sv * w_v[pl.ds(j * D, LANES)] + \
                    c_v[pl.ds(j * D, LANES)]
                outb[p][pl.ds(o + LANES, LANES)] = sv * w_v[pl.ds(j * D + LANES, LANES)] + \
                    c_v[pl.ds(j * D + LANES, LANES)]
            return cy
        lax.fori_loop(0, NB, nc_body, 0)

        for g in range(GGRP):
            pltpu.make_async_copy(table_hbm.at[xcat[p].at[pl.ds(g * GIDX, GIDX)]],
                                  gat[p].at[pl.ds(g * GIDX, GIDX), :], sem_g[p]).wait()

        def cat_body(b, cy):
            o_row = b * ROW_ELEMS + D
            r_row = b * N_CAT
            for v in range(VPR):
                r = r_row + v // 2
                k = (v % 2) * LANES
                outb[p][pl.ds(o_row + v * LANES, LANES)] = \
                    gat[p][r, pl.ds(k, LANES)] + pos_v[pl.ds(D + v * LANES, LANES)]
            return cy
        lax.fori_loop(0, NB, cat_body, 0)

        base = row0 + c * NB
        pltpu.async_copy(outb[p], out_hbm.at[pl.ds(base * ROW_ELEMS, OUT_PER_CHUNK)],
                         sem_o[p])

    start_in(0, 0)
    fire_gather(0)
    start_in(1, 1)

    def loop_body(cc, carry):
        for pp in (0, 1):
            c = cc * 2 + pp
            nxt = 1 - pp

            if pp == 0:
                fire_gather(nxt)
            else:
                @pl.when(cc < NCHUNK // 2 - 1)
                def _():
                    fire_gather(nxt)

            finish(c, pp, wait_out=cc >= 1)

            @pl.when(cc < NCHUNK // 2 - 1)
            def _():
                start_in(c + 2, pp)
        return carry

    lax.fori_loop(0, NCHUNK // 2, loop_body, 0)

    for p in (0, 1):
        pltpu.make_async_copy(outb[p], out_hbm.at[pl.ds(0, OUT_PER_CHUNK)],
                              sem_o[p]).wait()


@jax.jit
def kernel(x_cat, x_num, cat_tables, num_W, num_b, feature_pos, cls):
    mesh = plsc.VectorSubcoreMesh(core_axis_name="c", subcore_axis_name="s")
    k = pl.kernel(
        _sc_body,
        out_type=jax.ShapeDtypeStruct((B * ROW_ELEMS,), jnp.float32),
        mesh=mesh,
        compiler_params=pltpu.CompilerParams(use_tc_tiling_on_sc=False),
        scratch_types=[
            pltpu.VMEM((IDX_PER_CHUNK,), jnp.int32),
            pltpu.VMEM((IDX_PER_CHUNK,), jnp.int32),
            pltpu.VMEM((IDX_PER_CHUNK, D), jnp.float32),
            pltpu.VMEM((IDX_PER_CHUNK, D), jnp.float32),
            pltpu.VMEM((OUT_PER_CHUNK,), jnp.float32),
            pltpu.VMEM((OUT_PER_CHUNK,), jnp.float32),
            pltpu.VMEM((NB * N_NUM + LANES,), jnp.float32),
            pltpu.VMEM((NB * N_NUM + LANES,), jnp.float32),
            pltpu.VMEM((L_TOK * D,), jnp.float32),
            pltpu.VMEM((N_NUM * D,), jnp.float32),
            pltpu.VMEM((N_NUM * D,), jnp.float32),
            pltpu.VMEM((D,), jnp.float32),
            pltpu.VMEM((IDX_PER_CHUNK,), jnp.int32),
            pltpu.SemaphoreType.DMA,
            pltpu.SemaphoreType.DMA,
            pltpu.SemaphoreType.DMA,
            pltpu.SemaphoreType.DMA,
            pltpu.SemaphoreType.DMA,
            pltpu.SemaphoreType.DMA,
        ],
    )
    out = k(
        x_cat.reshape(B * N_CAT),
        x_num.reshape(B * N_NUM),
        cat_tables.reshape(N_CAT * VOCAB, D),
        num_W.reshape(N_NUM * D),
        num_b.reshape(N_NUM * D),
        feature_pos.reshape(L_TOK * D),
        cls.reshape(D),
    )
    return out.reshape(B, L_TOK, D)
